# baseline (speedup 1.0000x reference)
import jax
import jax.numpy as jnp
from jax import lax
from jax.experimental import pallas as pl
from jax.experimental.pallas import tpu as pltpu
from jax.experimental.pallas import tpu_sc as plsc

B, N, Q = 2048, 8192, 2048
L = 16
NC, NS = 2, 16
NW = NC * NS
ROWS_PER_W = B // NW
NBUF = 4


def _tec_body(x_hbm, y_hbm, out_hbm, xv, idxv, wv, rowbuf, outbuf, *sems):
    sems_in = sems[:NBUF]
    sems_out = sems[NBUF:]
    wid = lax.axis_index("s") * NC + lax.axis_index("c")
    base_row = wid * ROWS_PER_W

    def in_copy(b, row):
        return pltpu.make_async_copy(
            y_hbm.at[row], rowbuf.at[pl.ds(b * N, N)], sems_in[b])

    def out_copy(b, row):
        return pltpu.make_async_copy(
            outbuf.at[pl.ds(b * Q, Q)], out_hbm.at[row], sems_out[b])

    for b in range(NBUF):
        in_copy(b, base_row + b).start()

    pltpu.sync_copy(x_hbm, xv)
    scale = jnp.float32(N - 1)

    @plsc.parallel_loop(0, Q // L, unroll=4)
    def _idx_loop(i):
        x = xv[pl.ds(i * L, L)]
        t = x * scale
        idx = t.astype(jnp.int32)
        idx = jnp.minimum(jnp.maximum(idx, 0), N - 2)
        w = t - idx.astype(jnp.float32)
        idxv[pl.ds(i * L, L)] = idx
        wv[pl.ds(i * L, L)] = w

    def group_body(g, carry):
        row0 = base_row + g * NBUF
        for b in range(NBUF):
            in_copy(b, row0 + b).wait()

        @pl.when(g > 0)
        def _():
            for b in range(NBUF):
                out_copy(b, row0 - NBUF + b).wait()

        @plsc.parallel_loop(0, Q // L, unroll=4)
        def _q_loop(c):
            iv0 = idxv[pl.ds(c * L, L)]
            w = wv[pl.ds(c * L, L)]
            for b in range(NBUF):
                iv = iv0 + b * N if b else iv0
                y1 = plsc.load_gather(rowbuf, [iv])
                y2 = plsc.load_gather(rowbuf, [iv + 1])
                outbuf[pl.ds(b * Q + c * L, L)] = y1 + w * (y2 - y1)

        for b in range(NBUF):
            out_copy(b, row0 + b).start()

        @pl.when(g + 1 < ROWS_PER_W // NBUF)
        def _():
            for b in range(NBUF):
                in_copy(b, row0 + NBUF + b).start()
        return carry

    lax.fori_loop(0, ROWS_PER_W // NBUF, group_body, 0)

    for b in range(NBUF):
        out_copy(b, base_row + ROWS_PER_W - NBUF + b).wait()


def kernel(x_new_, y_points):
    mesh = plsc.VectorSubcoreMesh(core_axis_name="c", subcore_axis_name="s")
    k = pl.kernel(
        _tec_body,
        out_type=jax.ShapeDtypeStruct((B, Q), jnp.float32),
        mesh=mesh,
        compiler_params=pltpu.CompilerParams(needs_layout_passes=False),
        scratch_types=[
            pltpu.VMEM((Q,), jnp.float32),
            pltpu.VMEM((Q,), jnp.int32),
            pltpu.VMEM((Q,), jnp.float32),
            pltpu.VMEM((NBUF * N,), jnp.float32),
            pltpu.VMEM((NBUF * Q,), jnp.float32),
        ] + [pltpu.SemaphoreType.DMA] * (2 * NBUF),
    )
    return k(x_new_, y_points)


# speedup vs baseline: 1.3968x; 1.3968x over previous
import jax
import jax.numpy as jnp
from jax import lax
from jax.experimental import pallas as pl
from jax.experimental.pallas import tpu as pltpu
from jax.experimental.pallas import tpu_sc as plsc

B, N, Q = 2048, 8192, 2048
L = 16
NC, NS = 2, 16
NW = NC * NS
ROWS_PER_W = B // NW
NBUF = 4


def _tec_body(x_hbm, y_hbm, out_hbm, xv, idxv, wv, rowbuf, outbuf, *sems):
    sems_in = sems[:NBUF]
    sems_out = sems[NBUF:]
    wid = lax.axis_index("s") * NC + lax.axis_index("c")
    base_row = wid * ROWS_PER_W

    def in_copy(b, row):
        return pltpu.make_async_copy(
            y_hbm.at[row], rowbuf.at[pl.ds(b * N, N)], sems_in[b])

    def out_copy(b, row):
        return pltpu.make_async_copy(
            outbuf.at[pl.ds(b * Q, Q)], out_hbm.at[row], sems_out[b])

    for b in range(NBUF):
        in_copy(b, base_row + b).start()

    pltpu.sync_copy(x_hbm, xv)
    scale = jnp.float32(N - 1)

    @plsc.parallel_loop(0, Q // L, unroll=4)
    def _idx_loop(i):
        x = xv[pl.ds(i * L, L)]
        t = x * scale
        idx = t.astype(jnp.int32)
        idx = jnp.minimum(jnp.maximum(idx, 0), N - 2)
        w = t - idx.astype(jnp.float32)
        idxv[pl.ds(i * L, L)] = idx
        wv[pl.ds(i * L, L)] = w

    def group_body(g, carry):
        for b in range(NBUF):
            r = g * NBUF + b
            row = base_row + r
            in_copy(b, row).wait()

            @pl.when(g > 0)
            def _():
                out_copy(b, row - NBUF).wait()

            boff = b * N

            @plsc.parallel_loop(0, Q // L, unroll=8)
            def _q_loop(c):
                iv = idxv[pl.ds(c * L, L)] + boff
                w = wv[pl.ds(c * L, L)]
                y1 = plsc.load_gather(rowbuf, [iv])
                y2 = plsc.load_gather(rowbuf, [iv + 1])
                outbuf[pl.ds(b * Q + c * L, L)] = y1 + w * (y2 - y1)

            out_copy(b, row).start()

            @pl.when(r + NBUF < ROWS_PER_W)
            def _():
                in_copy(b, row + NBUF).start()
        return carry

    lax.fori_loop(0, ROWS_PER_W // NBUF, group_body, 0)

    for b in range(NBUF):
        out_copy(b, base_row + ROWS_PER_W - NBUF + b).wait()


def kernel(x_new_, y_points):
    mesh = plsc.VectorSubcoreMesh(core_axis_name="c", subcore_axis_name="s")
    k = pl.kernel(
        _tec_body,
        out_type=jax.ShapeDtypeStruct((B, Q), jnp.float32),
        mesh=mesh,
        compiler_params=pltpu.CompilerParams(needs_layout_passes=False),
        scratch_types=[
            pltpu.VMEM((Q,), jnp.float32),
            pltpu.VMEM((Q,), jnp.int32),
            pltpu.VMEM((Q,), jnp.float32),
            pltpu.VMEM((NBUF * N,), jnp.float32),
            pltpu.VMEM((NBUF * Q,), jnp.float32),
        ] + [pltpu.SemaphoreType.DMA] * (2 * NBUF),
    )
    return k(x_new_, y_points)
